# baseline (speedup 1.0000x reference)
import functools

import jax
import jax.numpy as jnp
from jax import lax
from jax.experimental import pallas as pl
from jax.experimental.pallas import tpu as pltpu
from jax.experimental.pallas import tpu_sc as plsc

_B, _C, _H, _W = 4, 96, 384, 384
_NW = 32
_ROWS = _H // _NW
_SLAB = _ROWS * _W
_CB = 8
_NCB = _C // _CB
_NV = _SLAB // 16


def _issue(x_hbm, xbuf, sem, base, cb, b, p):
    cps = []
    for k in range(_CB):
        n = b * _C + cb * _CB + k
        cps.append(pltpu.async_copy(
            x_hbm.at[n, pl.ds(base, _SLAB)], xbuf.at[p, k], sem))
    return cps


def _sc_body(x_hbm, m_hbm, out_hbm, mraw, mbuf, xbuf, orow, sem_x):
    cid = lax.axis_index("c")
    sid = lax.axis_index("s")
    wid = sid * 2 + cid
    base = wid * _SLAB

    for b in range(_B):
        pltpu.sync_copy(m_hbm.at[b, pl.ds(base, _SLAB)], mraw.at[b])

    def cvt_body(i, cacc):
        b = i // _NV
        j = i % _NV
        mv = mraw[b, pl.ds(j * 16, 16)].astype(jnp.float32)
        mbuf[b, pl.ds(j * 16, 16)] = mv
        return cacc + mv

    cacc = lax.fori_loop(0, _B * _NV, cvt_body, jnp.zeros((16,), jnp.float32))
    orow[192, :] = cacc

    handles = _issue(x_hbm, xbuf, sem_x, base, 0, 0, 0)
    for cb in range(_NCB):
        accs = tuple(jnp.zeros((16,), jnp.float32) for _ in range(2 * _CB))
        for b in range(_B):
            t = cb * _B + b
            p = t & 1
            nhandles = None
            if t + 1 < _NCB * _B:
                ncb, nb = divmod(t + 1, _B)
                nhandles = _issue(x_hbm, xbuf, sem_x, base, ncb, nb, (t + 1) & 1)
            for hnd in handles:
                hnd.wait()

            def jbody(j, carry, b=b, p=p):
                m = mbuf[b, pl.ds(j * 16, 16)]
                new1 = []
                new0 = []
                for k in range(_CB):
                    xv = xbuf[p, k, pl.ds(j * 16, 16)]
                    new1.append(carry[k] + xv * m)
                    new0.append(carry[_CB + k] + xv)
                return tuple(new1) + tuple(new0)

            accs = lax.fori_loop(0, _NV, jbody, accs)
            handles = nhandles
        for k in range(_CB):
            orow[cb * _CB + k, :] = accs[k]
            orow[96 + cb * _CB + k, :] = accs[_CB + k]

    pltpu.sync_copy(orow, out_hbm.at[wid])


_sc_pool = functools.partial(
    pl.kernel,
    out_type=jax.ShapeDtypeStruct((_NW, 200, 16), jnp.float32),
    mesh=plsc.VectorSubcoreMesh(
        core_axis_name="c", subcore_axis_name="s",
        num_cores=2, num_subcores=16),
    scratch_types=[
        pltpu.VMEM((_B, _SLAB), jnp.int32),
        pltpu.VMEM((_B, _SLAB), jnp.float32),
        pltpu.VMEM((2, _CB, _SLAB), jnp.float32),
        pltpu.VMEM((200, 16), jnp.float32),
        pltpu.SemaphoreType.DMA,
    ],
    compiler_params=pltpu.CompilerParams(use_tc_tiling_on_sc=False),
)(_sc_body)


def kernel(x, mask):
    B, C, H, W = x.shape
    x2 = x.reshape(B * C, H * W)
    m2 = mask.reshape(B, H * W)
    out = _sc_pool(x2, m2)
    s1 = jnp.sum(out[:, :_C, :], axis=(0, 2))
    s0 = jnp.sum(out[:, _C:2 * _C, :], axis=(0, 2))
    cnt = jnp.sum(out[:, 192, :])
    n_tot = jnp.float32(B * H * W)
    ch = s1 / cnt
    unch = (s0 - s1) / (n_tot - cnt)
    return (unch, ch)


# speedup vs baseline: 1.1171x; 1.1171x over previous
import functools

import jax
import jax.numpy as jnp
from jax import lax
from jax.experimental import pallas as pl
from jax.experimental.pallas import tpu as pltpu
from jax.experimental.pallas import tpu_sc as plsc

_B, _C, _H, _W = 4, 96, 384, 384
_H_TC = 320
_HT = 64
_NW = 32
_ROWS = (_H - _H_TC) // _NW
_OFF = _H_TC * _W
_SLAB = _ROWS * _W
_CB = 8
_NCB = _C // _CB
_NV = _SLAB // 16


def _tc_body(x_ref, m_ref, sums_ref, cnt_ref):
    b = pl.program_id(0)
    h = pl.program_id(1)

    @pl.when((b == 0) & (h == 0))
    def _init():
        sums_ref[...] = jnp.zeros_like(sums_ref)
        cnt_ref[0, 0] = jnp.float32(0.0)

    xb = x_ref[0]
    mb = m_ref[0].astype(jnp.float32)
    s1 = jnp.sum(xb * mb[None, :, :], axis=(1, 2))
    s0 = jnp.sum(xb, axis=(1, 2))
    sums_ref[...] += jnp.stack([s1, s0])
    cnt_ref[0, 0] += jnp.sum(mb)


def _tc_pool(x, mask):
    B, C, H, W = x.shape
    return pl.pallas_call(
        _tc_body,
        grid=(B, _H_TC // _HT),
        in_specs=[
            pl.BlockSpec((1, C, _HT, W), lambda b, h: (b, 0, h, 0)),
            pl.BlockSpec((1, _HT, W), lambda b, h: (b, h, 0)),
        ],
        out_specs=[
            pl.BlockSpec((2, C), lambda b, h: (0, 0)),
            pl.BlockSpec(memory_space=pltpu.SMEM),
        ],
        out_shape=[
            jax.ShapeDtypeStruct((2, C), jnp.float32),
            jax.ShapeDtypeStruct((1, 1), jnp.float32),
        ],
    )(x, mask)


def _issue(x_hbm, xbuf, sem, base, cb, b, p):
    cps = []
    for k in range(_CB):
        n = b * _C + cb * _CB + k
        cps.append(pltpu.async_copy(
            x_hbm.at[n, pl.ds(base, _SLAB)], xbuf.at[p, k], sem))
    return cps


def _sc_body(x_hbm, m_hbm, out_hbm, mraw, mbuf, xbuf, orow, sem_x):
    cid = lax.axis_index("c")
    sid = lax.axis_index("s")
    wid = sid * 2 + cid
    base = _OFF + wid * _SLAB

    for b in range(_B):
        pltpu.sync_copy(m_hbm.at[b, pl.ds(base, _SLAB)], mraw.at[b])

    def cvt_body(i, cacc):
        b = i // _NV
        j = i % _NV
        mv = mraw[b, pl.ds(j * 16, 16)].astype(jnp.float32)
        mbuf[b, pl.ds(j * 16, 16)] = mv
        return cacc + mv

    cacc = lax.fori_loop(0, _B * _NV, cvt_body, jnp.zeros((16,), jnp.float32))
    orow[192, :] = cacc

    handles = _issue(x_hbm, xbuf, sem_x, base, 0, 0, 0)
    for cb in range(_NCB):
        accs = tuple(jnp.zeros((16,), jnp.float32) for _ in range(2 * _CB))
        for b in range(_B):
            t = cb * _B + b
            p = t & 1
            nhandles = None
            if t + 1 < _NCB * _B:
                ncb, nb = divmod(t + 1, _B)
                nhandles = _issue(x_hbm, xbuf, sem_x, base, ncb, nb, (t + 1) & 1)
            for hnd in handles:
                hnd.wait()

            def jbody(j, carry, b=b, p=p):
                m = mbuf[b, pl.ds(j * 16, 16)]
                new1 = []
                new0 = []
                for k in range(_CB):
                    xv = xbuf[p, k, pl.ds(j * 16, 16)]
                    new1.append(carry[k] + xv * m)
                    new0.append(carry[_CB + k] + xv)
                return tuple(new1) + tuple(new0)

            accs = lax.fori_loop(0, _NV, jbody, accs)
            handles = nhandles
        for k in range(_CB):
            orow[cb * _CB + k, :] = accs[k]
            orow[96 + cb * _CB + k, :] = accs[_CB + k]

    pltpu.sync_copy(orow, out_hbm.at[wid])


_sc_pool = functools.partial(
    pl.kernel,
    out_type=jax.ShapeDtypeStruct((_NW, 200, 16), jnp.float32),
    mesh=plsc.VectorSubcoreMesh(
        core_axis_name="c", subcore_axis_name="s",
        num_cores=2, num_subcores=16),
    scratch_types=[
        pltpu.VMEM((_B, _SLAB), jnp.int32),
        pltpu.VMEM((_B, _SLAB), jnp.float32),
        pltpu.VMEM((2, _CB, _SLAB), jnp.float32),
        pltpu.VMEM((200, 16), jnp.float32),
        pltpu.SemaphoreType.DMA,
    ],
    compiler_params=pltpu.CompilerParams(use_tc_tiling_on_sc=False),
)(_sc_body)


def kernel(x, mask):
    B, C, H, W = x.shape
    x2 = x.reshape(B * C, H * W)
    m2 = mask.reshape(B, H * W)
    sc_out = _sc_pool(x2, m2)
    tc_sums, tc_cnt = _tc_pool(x, mask)
    s1 = tc_sums[0] + jnp.sum(sc_out[:, :_C, :], axis=(0, 2))
    s0 = tc_sums[1] + jnp.sum(sc_out[:, _C:2 * _C, :], axis=(0, 2))
    cnt = tc_cnt[0, 0] + jnp.sum(sc_out[:, 192, :])
    n_tot = jnp.float32(B * H * W)
    ch = s1 / cnt
    unch = (s0 - s1) / (n_tot - cnt)
    return (unch, ch)


# speedup vs baseline: 4.7044x; 4.2113x over previous
import jax
import jax.numpy as jnp
from jax.experimental import pallas as pl
from jax.experimental.pallas import tpu as pltpu

_HT = 96


def _pool_body(x_ref, m_ref, sums_ref, cnt_ref):
    b = pl.program_id(0)
    h = pl.program_id(1)

    @pl.when((b == 0) & (h == 0))
    def _init():
        sums_ref[...] = jnp.zeros_like(sums_ref)
        cnt_ref[0, 0] = jnp.float32(0.0)

    xb = x_ref[0]
    mb = m_ref[0].astype(jnp.float32)
    s1 = jnp.sum(xb * mb[None, :, :], axis=(1, 2))
    s0 = jnp.sum(xb, axis=(1, 2))
    sums_ref[...] += jnp.stack([s1, s0])
    cnt_ref[0, 0] += jnp.sum(mb)


def kernel(x, mask):
    B, C, H, W = x.shape
    grid = (B, H // _HT)
    sums, cnt = pl.pallas_call(
        _pool_body,
        grid=grid,
        in_specs=[
            pl.BlockSpec((1, C, _HT, W), lambda b, h: (b, 0, h, 0)),
            pl.BlockSpec((1, _HT, W), lambda b, h: (b, h, 0)),
        ],
        out_specs=[
            pl.BlockSpec((2, C), lambda b, h: (0, 0)),
            pl.BlockSpec(memory_space=pltpu.SMEM),
        ],
        out_shape=[
            jax.ShapeDtypeStruct((2, C), jnp.float32),
            jax.ShapeDtypeStruct((1, 1), jnp.float32),
        ],
    )(x, mask)
    n_ch = cnt[0, 0]
    n_tot = jnp.float32(B * H * W)
    ch = sums[0] / n_ch
    unch = (sums[1] - sums[0]) / (n_tot - n_ch)
    return (unch, ch)


# speedup vs baseline: 4.9365x; 1.0493x over previous
import jax
import jax.numpy as jnp
from jax.experimental import pallas as pl
from jax.experimental.pallas import tpu as pltpu

_HT = 64


def _pool_body(x_ref, m_ref, unch_ref, ch_ref, sums_ref, cnt_ref):
    b = pl.program_id(0)
    h = pl.program_id(1)

    @pl.when((b == 0) & (h == 0))
    def _init():
        sums_ref[...] = jnp.zeros_like(sums_ref)
        cnt_ref[0, 0] = jnp.float32(0.0)

    xb = x_ref[0]
    mb = m_ref[0].astype(jnp.float32)
    s1 = jnp.sum(xb * mb[None, :, :], axis=(1, 2))
    s0 = jnp.sum(xb, axis=(1, 2))
    sums_ref[...] += jnp.stack([s1, s0])
    cnt_ref[0, 0] += jnp.sum(mb)

    @pl.when((b == pl.num_programs(0) - 1) & (h == pl.num_programs(1) - 1))
    def _finish():
        n_ch = cnt_ref[0, 0]
        n_tot = jnp.float32(m_ref.shape[1] * m_ref.shape[2]
                            * pl.num_programs(0) * pl.num_programs(1))
        tot1 = sums_ref[0, :]
        tot0 = sums_ref[1, :]
        ch_ref[0, :] = tot1 / n_ch
        unch_ref[0, :] = (tot0 - tot1) / (n_tot - n_ch)


def kernel(x, mask):
    B, C, H, W = x.shape
    grid = (B, H // _HT)
    unch, ch = pl.pallas_call(
        _pool_body,
        grid=grid,
        in_specs=[
            pl.BlockSpec((1, C, _HT, W), lambda b, h: (b, 0, h, 0)),
            pl.BlockSpec((1, _HT, W), lambda b, h: (b, h, 0)),
        ],
        out_specs=[
            pl.BlockSpec((1, C), lambda b, h: (0, 0)),
            pl.BlockSpec((1, C), lambda b, h: (0, 0)),
        ],
        out_shape=[
            jax.ShapeDtypeStruct((1, C), jnp.float32),
            jax.ShapeDtypeStruct((1, C), jnp.float32),
        ],
        scratch_shapes=[
            pltpu.VMEM((2, C), jnp.float32),
            pltpu.SMEM((1, 1), jnp.float32),
        ],
    )(x, mask)
    return (unch.reshape(C), ch.reshape(C))
